# baseline (speedup 1.0000x reference)
import jax
import jax.numpy as jnp
from jax.experimental import pallas as pl
from jax.experimental.pallas import tpu as pltpu

_W = 0.1
_NBINS = 64
_EPS = 1e-08
_LOG2E = 1.4426950408889634


def _kl_body(pred_ref, targ_ref, out_ref, hist_p, hist_t):
    t = targ_ref[0]
    p = pred_ref[0]
    rows = t.shape[0]
    ch = 64
    nch = rows // ch

    vmin = jnp.min(t)
    vmax = jnp.max(t)
    denom = vmax - vmin + _EPS
    w = 1.0 / _NBINS
    inv_denom = 1.0 / denom
    s2 = jnp.float32(_LOG2E / (2.0 * w * w)) * inv_denom * inv_denom
    ns2 = -s2
    step = denom * w

    def bin_body(b, _):
        m = vmin + (b.astype(jnp.float32) + 0.5) * step
        acc_t = jnp.zeros((ch, 128), jnp.float32)
        acc_p = jnp.zeros((ch, 128), jnp.float32)
        for i in range(nch):
            tc = targ_ref[0, i * ch:(i + 1) * ch, :]
            pc = pred_ref[0, i * ch:(i + 1) * ch, :]
            ut = tc - m
            acc_t = acc_t + jnp.exp2(ut * (ut * ns2))
            up = pc - m
            acc_p = acc_p + jnp.exp2(up * (up * ns2))
        hist_t[b] = acc_t
        hist_p[b] = acc_p
        return 0

    jax.lax.fori_loop(0, _NBINS, bin_body, 0, unroll=2)

    ht = jnp.sum(jnp.sum(hist_t[...], axis=1), axis=1, keepdims=True)
    hp = jnp.sum(jnp.sum(hist_p[...], axis=1), axis=1, keepdims=True)
    tp = ht / (jnp.sum(ht) + _EPS)
    pp = hp / (jnp.sum(hp) + _EPS)
    kl = jnp.sum(tp * (jnp.log(tp + _EPS) - jnp.log(pp + _EPS)))
    out_ref[0] = jnp.full((8, 128), kl, dtype=jnp.float32)


def _kl_pallas(p3, t3):
    b, rows, lanes = p3.shape
    return pl.pallas_call(
        _kl_body,
        out_shape=jax.ShapeDtypeStruct((b, 8, 128), jnp.float32),
        grid=(b,),
        in_specs=[
            pl.BlockSpec((1, rows, lanes), lambda i: (i, 0, 0)),
            pl.BlockSpec((1, rows, lanes), lambda i: (i, 0, 0)),
        ],
        out_specs=pl.BlockSpec((1, 8, 128), lambda i: (i, 0, 0)),
        scratch_shapes=[
            pltpu.VMEM((_NBINS, 64, 128), jnp.float32),
            pltpu.VMEM((_NBINS, 64, 128), jnp.float32),
        ],
        compiler_params=pltpu.CompilerParams(
            dimension_semantics=("parallel",),
        ),
        name="kl_soft_hist",
    )(p3, t3)


def kernel(pred, target):
    B = pred.shape[0]
    n = pred.size // B
    lanes = 128
    rows = n // lanes
    p3 = pred.reshape(B, rows, lanes)
    t3 = target.reshape(B, rows, lanes)

    out = _kl_pallas(p3, t3)

    return _W * jnp.mean(out[:, 0, 0])
